# baseline (speedup 1.0000x reference)
import functools

import jax
import jax.numpy as jnp
from jax.experimental import pallas as pl

B = 4
N = 2048
M = 256
K = 8
CENC = 60
L = 10
EPS = 1e-5

NB_A = 256
PB_B = 8192
NB_C = 256
PB_C = NB_C * K
P_TOT = B * N * K

_F32 = jnp.float32
_BF16 = jnp.bfloat16
_INVP = 1.0 / P_TOT


def _dot(x, w):
    return jax.lax.dot_general(x, w, (((1,), (1,)), ((), ())),
                               preferred_element_type=_F32)


def _gather_dot(oh, w):
    hi = w.astype(_BF16).astype(_F32)
    r = w - hi
    mid = r.astype(_BF16).astype(_F32)
    lo = r - mid
    return _dot(oh, hi) + _dot(oh, mid) + _dot(oh, lo)


def _eye(n):
    r = jax.lax.broadcasted_iota(jnp.int32, (n, n), 0)
    c = jax.lax.broadcasted_iota(jnp.int32, (n, n), 1)
    return (r == c).astype(_F32)


def _transpose(x):
    return _gather_dot(_eye(x.shape[1]), x)


_INV2PI = 0.15915493667125702
_C1 = 6.28125
_C2 = 0.0019353071693331003
_C3 = 1.0253131677018246e-11
_SIN_COEF = (0.9999995827674866, -0.1666654646396637, 0.008332370780408382,
             -0.00019807845819741488, 2.69886936621333e-06,
             -2.03291836697872e-08)


def _fast_sin(x):
    n = jnp.floor(x * _INV2PI + 0.5)
    r = ((x - n * _C1) - n * _C2) - n * _C3
    u = r * r
    p = _F32(_SIN_COEF[5])
    for c in (_SIN_COEF[4], _SIN_COEF[3], _SIN_COEF[2], _SIN_COEF[1],
              _SIN_COEF[0]):
        p = p * u + c
    return r * p


def _posenc(knn, freqs, phase):
    P_ = knn.shape[0]
    xb = jnp.concatenate(
        [jnp.broadcast_to(knn[:, c:c + 1], (P_, 20)) for c in range(3)], axis=1)
    xf = xb * freqs
    return _fast_sin(xf + phase)


def _enc_consts(dtype):
    j = jax.lax.broadcasted_iota(jnp.int32, (1, CENC), 1)
    freqs = jnp.round(jnp.exp(0.6931471805599453 * ((j % 20) % L).astype(dtype)))
    phase = jnp.where((j % 20) >= L, dtype(1.5707963267948966), dtype(0.0))
    return freqs, phase


def _bn_fold(W, gram_in, mu_in, b_row, g_row, be_row):
    m0 = _dot(mu_in, W)
    t = _dot(gram_in, W)
    q = jnp.sum(_transpose(W) * t, axis=0, keepdims=True)
    var = q - m0 * m0
    mean = m0 + b_row
    s = g_row / jnp.sqrt(var + EPS)
    c = (b_row - mean) * s + be_row
    return s, c


def _pass_a_kernel(posT_ref, seed_ref, pos256_ref,
                   W1_ref, b1_ref, g1_ref, be1_ref,
                   W5_ref, b5_ref, g5_ref, be5_ref,
                   rel_ref, penc_ref, relm_ref, pencm_ref,
                   s1_ref, c1_ref, s5_ref, c5_ref):
    b = pl.program_id(0)
    i = pl.program_id(1)
    q = posT_ref[0]
    s = seed_ref[0]
    p256 = pos256_ref[0]

    qb = q.astype(_BF16).astype(_F32)
    sb = s.astype(_BF16).astype(_F32)
    nq = (q[:, 0:1] * q[:, 0:1] + q[:, 1:2] * q[:, 1:2]) + q[:, 2:3] * q[:, 2:3]
    ns = (s[0:1, :] * s[0:1, :] + s[1:2, :] * s[1:2, :]) + s[2:3, :] * s[2:3, :]
    dot = ((qb[:, 0:1] * sb[0:1, :] + qb[:, 1:2] * sb[1:2, :])
           + qb[:, 2:3] * sb[2:3, :])
    d = (nq - 2.0 * dot) + ns

    iota = jax.lax.broadcasted_iota(jnp.int32, (NB_A, M), 1).astype(_F32)
    knn_parts = []
    for _ in range(K):
        mn = jnp.min(d, axis=1, keepdims=True)
        idx = jnp.min(jnp.where(d == mn, iota, _F32(M)), axis=1, keepdims=True)
        oh = (iota == idx).astype(_F32)
        knn_parts.append(_gather_dot(oh, p256))
        d = jnp.where(iota == idx, jnp.inf, d)

    knn = jnp.concatenate(knn_parts, axis=0)
    qt = jnp.concatenate([q] * K, axis=0)
    rel = qt - knn
    rel_ref[...] = rel

    freqs, phase = _enc_consts(_F32)
    penc = _posenc(knn, freqs, phase)
    penc_ref[...] = penc

    rel_sum = jnp.sum(rel, axis=0, keepdims=True)
    rel_gram = jax.lax.dot_general(rel, rel, (((0,), (0,)), ((), ())),
                                   preferred_element_type=_F32)
    penc_sum = jnp.sum(penc, axis=0, keepdims=True)
    penc_gram = jax.lax.dot_general(penc, penc, (((0,), (0,)), ((), ())),
                                    preferred_element_type=_F32)

    relm = jnp.concatenate([rel_sum, rel_gram], axis=0)
    pencm = jnp.concatenate([penc_sum, penc_gram], axis=0)

    @pl.when((b == 0) & (i == 0))
    def _():
        relm_ref[...] = relm
        pencm_ref[...] = pencm

    @pl.when((b > 0) | (i > 0))
    def _():
        relm_ref[...] += relm
        pencm_ref[...] += pencm

    @pl.when((b == B - 1) & (i == (N // NB_A) - 1))
    def _():
        s1, c1 = _bn_fold(W1_ref[...], relm_ref[1:4, :] * _INVP,
                          relm_ref[0:1, :] * _INVP,
                          b1_ref[...], g1_ref[...], be1_ref[...])
        s1_ref[...] = s1
        c1_ref[...] = c1
        s5, c5 = _bn_fold(W5_ref[...], pencm_ref[1:61, :] * _INVP,
                          pencm_ref[0:1, :] * _INVP,
                          b5_ref[...], g5_ref[...], be5_ref[...])
        s5_ref[...] = s5
        c5_ref[...] = c5


def _pass_b_kernel(rel_ref, W1_ref, s1_ref, c1_ref,
                   W2_ref, b2_ref, W3_ref, b3_ref, g3_ref, be3_ref,
                   a1m_ref, s3_ref, c3_ref):
    i = pl.program_id(0)
    a1 = jnp.maximum(_dot(rel_ref[...], W1_ref[...]) * s1_ref[...]
                     + c1_ref[...], 0.0)
    a1_sum = jnp.sum(a1, axis=0, keepdims=True)
    a1_gram = jax.lax.dot_general(a1, a1, (((0,), (0,)), ((), ())),
                                  preferred_element_type=_F32)
    a1m = jnp.concatenate([a1_sum, a1_gram], axis=0)

    @pl.when(i == 0)
    def _():
        a1m_ref[...] = a1m

    @pl.when(i > 0)
    def _():
        a1m_ref[...] += a1m

    @pl.when(i == (P_TOT // PB_B) - 1)
    def _():
        Mw = jax.lax.dot_general(W3_ref[...], W2_ref[...],
                                 (((1,), (0,)), ((), ())),
                                 preferred_element_type=_F32)
        b_row = _dot(b2_ref[...], W3_ref[...]) + b3_ref[...]
        s3, c3 = _bn_fold(Mw, a1m_ref[1:65, :] * _INVP,
                          a1m_ref[0:1, :] * _INVP,
                          b_row, g3_ref[...], be3_ref[...])
        s3_ref[...] = s3
        c3_ref[...] = c3


def _pass_c_kernel(rel_ref, penc_ref,
                   W1_ref, s1_ref, c1_ref, W2_ref, b2_ref,
                   W3_ref, s3_ref, c3_ref, W4_ref, b4_ref,
                   W5_ref, s5_ref, c5_ref, W6_ref, b6_ref,
                   out_ref):
    rel = rel_ref[...]
    penc = penc_ref[...]

    a1 = jnp.maximum(_dot(rel, W1_ref[...]) * s1_ref[...] + c1_ref[...], 0.0)
    pe = _dot(a1, W2_ref[...]) + b2_ref[...]
    w3 = jnp.maximum(_dot(pe, W3_ref[...]) * s3_ref[...] + c3_ref[...], 0.0)
    w4 = _dot(w3, W4_ref[...]) + b4_ref[...]

    f5 = jnp.maximum(_dot(penc, W5_ref[...]) * s5_ref[...] + c5_ref[...], 0.0)
    f6 = _dot(f5, W6_ref[...]) + b6_ref[...] + pe

    wk = [w4[k * NB_C:(k + 1) * NB_C, :] for k in range(K)]
    mx = wk[0]
    for k in range(1, K):
        mx = jnp.maximum(mx, wk[k])
    ek = [jnp.exp(wk[k] - mx) for k in range(K)]
    den = ek[0]
    for k in range(1, K):
        den = den + ek[k]
    acc = jnp.zeros((NB_C, 128), _F32)
    for k in range(K):
        acc = acc + ek[k] * f6[k * NB_C:(k + 1) * NB_C, :]
    out_ref[0] = _transpose(acc / den)


@functools.partial(jax.jit, static_argnums=())
def kernel(pos, seed, W1, b1, g1, be1, W2, b2, W3, b3, g3, be3, W4, b4,
           W5, b5, g5, be5, W6, b6):
    nbn = N // NB_A
    posT = pos.transpose(0, 2, 1)
    pos256 = pos[:, :, :M]
    row = lambda v: v[None, :]

    full = lambda shape: pl.BlockSpec(shape, lambda b, i: tuple(0 for _ in shape))

    rel, penc, relm, pencm, s1, c1, s5, c5 = pl.pallas_call(
        _pass_a_kernel,
        grid=(B, nbn),
        in_specs=[
            pl.BlockSpec((1, NB_A, 3), lambda b, i: (b, i, 0)),
            pl.BlockSpec((1, 3, M), lambda b, i: (b, 0, 0)),
            pl.BlockSpec((1, 3, M), lambda b, i: (b, 0, 0)),
            full((64, 3)), full((1, 64)), full((1, 64)), full((1, 64)),
            full((128, CENC)), full((1, 128)), full((1, 128)), full((1, 128)),
        ],
        out_specs=[
            pl.BlockSpec((K * NB_A, 3), lambda b, i: (b * (N // NB_A) + i, 0)),
            pl.BlockSpec((K * NB_A, CENC), lambda b, i: (b * (N // NB_A) + i, 0)),
            full((4, 3)), full((61, CENC)),
            full((1, 64)), full((1, 64)), full((1, 128)), full((1, 128)),
        ],
        out_shape=[
            jax.ShapeDtypeStruct((P_TOT, 3), _F32),
            jax.ShapeDtypeStruct((P_TOT, CENC), _F32),
            jax.ShapeDtypeStruct((4, 3), _F32),
            jax.ShapeDtypeStruct((61, CENC), _F32),
            jax.ShapeDtypeStruct((1, 64), _F32),
            jax.ShapeDtypeStruct((1, 64), _F32),
            jax.ShapeDtypeStruct((1, 128), _F32),
            jax.ShapeDtypeStruct((1, 128), _F32),
        ],
    )(posT, seed, pos256, W1, row(b1), row(g1), row(be1),
      W5, row(b5), row(g5), row(be5))

    fullb = lambda shape: pl.BlockSpec(shape, lambda i: tuple(0 for _ in shape))
    a1m, s3, c3 = pl.pallas_call(
        _pass_b_kernel,
        grid=(P_TOT // PB_B,),
        in_specs=[
            pl.BlockSpec((PB_B, 3), lambda i: (i, 0)),
            fullb((64, 3)), fullb((1, 64)), fullb((1, 64)),
            fullb((128, 64)), fullb((1, 128)),
            fullb((512, 128)), fullb((1, 512)), fullb((1, 512)), fullb((1, 512)),
        ],
        out_specs=[fullb((65, 64)), fullb((1, 512)), fullb((1, 512))],
        out_shape=[
            jax.ShapeDtypeStruct((65, 64), _F32),
            jax.ShapeDtypeStruct((1, 512), _F32),
            jax.ShapeDtypeStruct((1, 512), _F32),
        ],
    )(rel, W1, s1, c1, W2, row(b2), W3, row(b3), g3[None, :], be3[None, :])

    out = pl.pallas_call(
        _pass_c_kernel,
        grid=(P_TOT // PB_C,),
        in_specs=[
            pl.BlockSpec((PB_C, 3), lambda i: (i, 0)),
            pl.BlockSpec((PB_C, CENC), lambda i: (i, 0)),
            fullb((64, 3)), fullb((1, 64)), fullb((1, 64)),
            fullb((128, 64)), fullb((1, 128)),
            fullb((512, 128)), fullb((1, 512)), fullb((1, 512)),
            fullb((128, 512)), fullb((1, 128)),
            fullb((128, CENC)), fullb((1, 128)), fullb((1, 128)),
            fullb((128, 128)), fullb((1, 128)),
        ],
        out_specs=pl.BlockSpec((1, 128, NB_C),
                               lambda i: (i // (N // NB_C), 0, i % (N // NB_C))),
        out_shape=jax.ShapeDtypeStruct((B, 128, N), _F32),
    )(rel, penc, W1, s1, c1, W2, row(b2), W3, s3, c3, W4, row(b4),
      W5, s5, c5, W6, row(b6))

    return out


# speedup vs baseline: 1.0627x; 1.0627x over previous
import functools

import jax
import jax.numpy as jnp
from jax.experimental import pallas as pl

B = 4
N = 2048
M = 256
K = 8
CENC = 60
L = 10
EPS = 1e-5

NB_A = 256
PB_B = 8192
NB_C = 256
PB_C = NB_C * K
P_TOT = B * N * K

_F32 = jnp.float32
_BF16 = jnp.bfloat16
_INVP = 1.0 / P_TOT


def _dot(x, w):
    return jax.lax.dot_general(x, w, (((1,), (1,)), ((), ())),
                               preferred_element_type=_F32)


def _gather_dot(oh, w):
    hi = w.astype(_BF16).astype(_F32)
    r = w - hi
    mid = r.astype(_BF16).astype(_F32)
    lo = r - mid
    return _dot(oh, hi) + _dot(oh, mid) + _dot(oh, lo)


def _eye(n):
    r = jax.lax.broadcasted_iota(jnp.int32, (n, n), 0)
    c = jax.lax.broadcasted_iota(jnp.int32, (n, n), 1)
    return (r == c).astype(_F32)


def _transpose(x):
    return _gather_dot(_eye(x.shape[1]), x)


_INV2PI = 0.15915493667125702
_C1 = 6.28125
_C2 = 0.0019353071693331003
_C3 = 1.0253131677018246e-11
_SIN_COEF = (0.9999995827674866, -0.1666654646396637, 0.008332370780408382,
             -0.00019807845819741488, 2.69886936621333e-06,
             -2.03291836697872e-08)


def _fast_sin(x):
    n = jnp.floor(x * _INV2PI + 0.5)
    r = ((x - n * _C1) - n * _C2) - n * _C3
    u = r * r
    p = _F32(_SIN_COEF[5])
    for c in (_SIN_COEF[4], _SIN_COEF[3], _SIN_COEF[2], _SIN_COEF[1],
              _SIN_COEF[0]):
        p = p * u + c
    return r * p


def _posenc(knn, freqs, phase):
    P_ = knn.shape[0]
    xb = jnp.concatenate(
        [jnp.broadcast_to(knn[:, c:c + 1], (P_, 20)) for c in range(3)], axis=1)
    xf = xb * freqs
    return _fast_sin(xf + phase)


def _enc_consts(dtype):
    j = jax.lax.broadcasted_iota(jnp.int32, (1, CENC), 1)
    freqs = jnp.round(jnp.exp(0.6931471805599453 * ((j % 20) % L).astype(dtype)))
    phase = jnp.where((j % 20) >= L, dtype(1.5707963267948966), dtype(0.0))
    return freqs, phase


def _bn_fold(W, gram_in, mu_in, b_row, g_row, be_row):
    m0 = _dot(mu_in, W)
    t = _dot(gram_in, W)
    q = jnp.sum(_transpose(W) * t, axis=0, keepdims=True)
    var = q - m0 * m0
    mean = m0 + b_row
    s = g_row / jnp.sqrt(var + EPS)
    c = (b_row - mean) * s + be_row
    return s, c


def _pass_a_kernel(posT_ref, seed_ref, pos256_ref,
                   W1_ref, b1_ref, g1_ref, be1_ref,
                   W5_ref, b5_ref, g5_ref, be5_ref,
                   rel_ref, penc_ref, relm_ref, pencm_ref,
                   W1s_ref, c1_ref, W5s_ref, c5_ref):
    b = pl.program_id(0)
    i = pl.program_id(1)
    q = posT_ref[0]
    s = seed_ref[0]
    p256 = pos256_ref[0]

    qb = q.astype(_BF16).astype(_F32)
    sb = s.astype(_BF16).astype(_F32)
    nq = (q[:, 0:1] * q[:, 0:1] + q[:, 1:2] * q[:, 1:2]) + q[:, 2:3] * q[:, 2:3]
    ns = (s[0:1, :] * s[0:1, :] + s[1:2, :] * s[1:2, :]) + s[2:3, :] * s[2:3, :]
    dot = ((qb[:, 0:1] * sb[0:1, :] + qb[:, 1:2] * sb[1:2, :])
           + qb[:, 2:3] * sb[2:3, :])
    d = (nq - 2.0 * dot) + ns

    iota = jax.lax.broadcasted_iota(jnp.int32, (NB_A, M), 1).astype(_F32)
    knn_parts = []
    for _ in range(K):
        mn = jnp.min(d, axis=1, keepdims=True)
        idx = jnp.min(jnp.where(d == mn, iota, _F32(M)), axis=1, keepdims=True)
        oh = (iota == idx).astype(_F32)
        knn_parts.append(_gather_dot(oh, p256))
        d = jnp.where(iota == idx, jnp.inf, d)

    knn = jnp.concatenate(knn_parts, axis=0)
    qt = jnp.concatenate([q] * K, axis=0)
    rel = qt - knn
    rel_ref[...] = rel

    freqs, phase = _enc_consts(_F32)
    penc = _posenc(knn, freqs, phase)
    penc_ref[...] = penc

    rel_sum = jnp.sum(rel, axis=0, keepdims=True)
    rel_gram = jax.lax.dot_general(rel, rel, (((0,), (0,)), ((), ())),
                                   preferred_element_type=_F32)
    penc_sum = jnp.sum(penc, axis=0, keepdims=True)
    penc_gram = jax.lax.dot_general(penc, penc, (((0,), (0,)), ((), ())),
                                    preferred_element_type=_F32)

    relm = jnp.concatenate([rel_sum, rel_gram], axis=0)
    pencm = jnp.concatenate([penc_sum, penc_gram], axis=0)

    @pl.when((b == 0) & (i == 0))
    def _():
        relm_ref[...] = relm
        pencm_ref[...] = pencm

    @pl.when((b > 0) | (i > 0))
    def _():
        relm_ref[...] += relm
        pencm_ref[...] += pencm

    @pl.when((b == B - 1) & (i == (N // NB_A) - 1))
    def _():
        s1, c1 = _bn_fold(W1_ref[...], relm_ref[1:4, :] * _INVP,
                          relm_ref[0:1, :] * _INVP,
                          b1_ref[...], g1_ref[...], be1_ref[...])
        W1s_ref[...] = W1_ref[...] * _transpose(s1)
        c1_ref[...] = c1
        s5, c5 = _bn_fold(W5_ref[...], pencm_ref[1:61, :] * _INVP,
                          pencm_ref[0:1, :] * _INVP,
                          b5_ref[...], g5_ref[...], be5_ref[...])
        W5s_ref[...] = W5_ref[...] * _transpose(s5)
        c5_ref[...] = c5


def _pass_b_kernel(rel_ref, W1s_ref, c1_ref,
                   W2_ref, b2_ref, W3_ref, b3_ref, g3_ref, be3_ref,
                   a1m_ref, W3s_ref, c3_ref):
    i = pl.program_id(0)
    a1 = jnp.maximum(_dot(rel_ref[...], W1s_ref[...]) + c1_ref[...], 0.0)
    a1_sum = jnp.sum(a1, axis=0, keepdims=True)
    a1_gram = jax.lax.dot_general(a1, a1, (((0,), (0,)), ((), ())),
                                  preferred_element_type=_F32)
    a1m = jnp.concatenate([a1_sum, a1_gram], axis=0)

    @pl.when(i == 0)
    def _():
        a1m_ref[...] = a1m

    @pl.when(i > 0)
    def _():
        a1m_ref[...] += a1m

    @pl.when(i == (P_TOT // PB_B) - 1)
    def _():
        Mw = jax.lax.dot_general(W3_ref[...], W2_ref[...],
                                 (((1,), (0,)), ((), ())),
                                 preferred_element_type=_F32)
        b_row = _dot(b2_ref[...], W3_ref[...]) + b3_ref[...]
        s3, c3 = _bn_fold(Mw, a1m_ref[1:65, :] * _INVP,
                          a1m_ref[0:1, :] * _INVP,
                          b_row, g3_ref[...], be3_ref[...])
        W3s_ref[...] = W3_ref[...] * _transpose(s3)
        c3_ref[...] = c3


def _pass_c_kernel(rel_ref, penc_ref,
                   W1s_ref, c1_ref, W2_ref, b2_ref,
                   W3s_ref, c3_ref, W4_ref, b4_ref,
                   W5s_ref, c5_ref, W6_ref, b6_ref,
                   out_ref):
    rel = rel_ref[...]
    penc = penc_ref[...]

    a1 = jnp.maximum(_dot(rel, W1s_ref[...]) + c1_ref[...], 0.0)
    pe = _dot(a1, W2_ref[...]) + b2_ref[...]
    w3 = jnp.maximum(_dot(pe, W3s_ref[...]) + c3_ref[...], 0.0)
    w4 = _dot(w3, W4_ref[...]) + b4_ref[...]

    f5 = jnp.maximum(_dot(penc, W5s_ref[...]) + c5_ref[...], 0.0)
    f6 = _dot(f5, W6_ref[...]) + b6_ref[...] + pe

    wk = [w4[k * NB_C:(k + 1) * NB_C, :] for k in range(K)]
    mx = wk[0]
    for k in range(1, K):
        mx = jnp.maximum(mx, wk[k])
    ek = [jnp.exp(wk[k] - mx) for k in range(K)]
    den = ek[0]
    for k in range(1, K):
        den = den + ek[k]
    acc = jnp.zeros((NB_C, 128), _F32)
    for k in range(K):
        acc = acc + ek[k] * f6[k * NB_C:(k + 1) * NB_C, :]
    out_ref[...] = acc / den


@functools.partial(jax.jit, static_argnums=())
def kernel(pos, seed, W1, b1, g1, be1, W2, b2, W3, b3, g3, be3, W4, b4,
           W5, b5, g5, be5, W6, b6):
    nbn = N // NB_A
    posT = pos.transpose(0, 2, 1)
    pos256 = pos[:, :, :M]
    row = lambda v: v[None, :]

    full = lambda shape: pl.BlockSpec(shape, lambda b, i: tuple(0 for _ in shape))

    rel, penc, relm, pencm, W1s, c1, W5s, c5 = pl.pallas_call(
        _pass_a_kernel,
        grid=(B, nbn),
        in_specs=[
            pl.BlockSpec((1, NB_A, 3), lambda b, i: (b, i, 0)),
            pl.BlockSpec((1, 3, M), lambda b, i: (b, 0, 0)),
            pl.BlockSpec((1, 3, M), lambda b, i: (b, 0, 0)),
            full((64, 3)), full((1, 64)), full((1, 64)), full((1, 64)),
            full((128, CENC)), full((1, 128)), full((1, 128)), full((1, 128)),
        ],
        out_specs=[
            pl.BlockSpec((K * NB_A, 3), lambda b, i: (b * (N // NB_A) + i, 0)),
            pl.BlockSpec((K * NB_A, CENC), lambda b, i: (b * (N // NB_A) + i, 0)),
            full((4, 3)), full((61, CENC)),
            full((64, 3)), full((1, 64)), full((128, CENC)), full((1, 128)),
        ],
        out_shape=[
            jax.ShapeDtypeStruct((P_TOT, 3), _F32),
            jax.ShapeDtypeStruct((P_TOT, CENC), _F32),
            jax.ShapeDtypeStruct((4, 3), _F32),
            jax.ShapeDtypeStruct((61, CENC), _F32),
            jax.ShapeDtypeStruct((64, 3), _F32),
            jax.ShapeDtypeStruct((1, 64), _F32),
            jax.ShapeDtypeStruct((128, CENC), _F32),
            jax.ShapeDtypeStruct((1, 128), _F32),
        ],
    )(posT, seed, pos256, W1, row(b1), row(g1), row(be1),
      W5, row(b5), row(g5), row(be5))

    fullb = lambda shape: pl.BlockSpec(shape, lambda i: tuple(0 for _ in shape))
    a1m, W3s, c3 = pl.pallas_call(
        _pass_b_kernel,
        grid=(P_TOT // PB_B,),
        in_specs=[
            pl.BlockSpec((PB_B, 3), lambda i: (i, 0)),
            fullb((64, 3)), fullb((1, 64)),
            fullb((128, 64)), fullb((1, 128)),
            fullb((512, 128)), fullb((1, 512)), fullb((1, 512)), fullb((1, 512)),
        ],
        out_specs=[fullb((65, 64)), fullb((512, 128)), fullb((1, 512))],
        out_shape=[
            jax.ShapeDtypeStruct((65, 64), _F32),
            jax.ShapeDtypeStruct((512, 128), _F32),
            jax.ShapeDtypeStruct((1, 512), _F32),
        ],
    )(rel, W1s, c1, W2, row(b2), W3, row(b3), row(g3), row(be3))

    out = pl.pallas_call(
        _pass_c_kernel,
        grid=(P_TOT // PB_C,),
        in_specs=[
            pl.BlockSpec((PB_C, 3), lambda i: (i, 0)),
            pl.BlockSpec((PB_C, CENC), lambda i: (i, 0)),
            fullb((64, 3)), fullb((1, 64)),
            fullb((128, 64)), fullb((1, 128)),
            fullb((512, 128)), fullb((1, 512)),
            fullb((128, 512)), fullb((1, 128)),
            fullb((128, CENC)), fullb((1, 128)),
            fullb((128, 128)), fullb((1, 128)),
        ],
        out_specs=pl.BlockSpec((NB_C, 128), lambda i: (i, 0)),
        out_shape=jax.ShapeDtypeStruct((B * N, 128), _F32),
    )(rel, penc, W1s, c1, W2, row(b2), W3s, c3, W4, row(b4),
      W5s, c5, W6, row(b6))

    return out.reshape(B, N, 128).transpose(0, 2, 1)


# speedup vs baseline: 1.1076x; 1.0423x over previous
import functools

import jax
import jax.numpy as jnp
from jax.experimental import pallas as pl

B = 4
N = 2048
M = 256
K = 8
CENC = 60
L = 10
EPS = 1e-5

NB_A = 512
PB_B = 8192
NB_C = 512
PB_C = NB_C * K
P_TOT = B * N * K

_F32 = jnp.float32
_BF16 = jnp.bfloat16
_INVP = 1.0 / P_TOT


def _dot(x, w):
    return jax.lax.dot_general(x, w, (((1,), (1,)), ((), ())),
                               preferred_element_type=_F32)


def _gather_dot(oh, w):
    hi = w.astype(_BF16).astype(_F32)
    r = w - hi
    mid = r.astype(_BF16).astype(_F32)
    lo = r - mid
    return _dot(oh, hi) + _dot(oh, mid) + _dot(oh, lo)


def _eye(n):
    r = jax.lax.broadcasted_iota(jnp.int32, (n, n), 0)
    c = jax.lax.broadcasted_iota(jnp.int32, (n, n), 1)
    return (r == c).astype(_F32)


def _transpose(x):
    return _gather_dot(_eye(x.shape[1]), x)


_INV2PI = 0.15915493667125702
_C1 = 6.28125
_C2 = 0.0019353071693331003
_C3 = 1.0253131677018246e-11
_SIN_COEF = (0.9999995827674866, -0.1666654646396637, 0.008332370780408382,
             -0.00019807845819741488, 2.69886936621333e-06,
             -2.03291836697872e-08)


def _fast_sin(x):
    n = jnp.floor(x * _INV2PI + 0.5)
    r = ((x - n * _C1) - n * _C2) - n * _C3
    u = r * r
    p = _F32(_SIN_COEF[5])
    for c in (_SIN_COEF[4], _SIN_COEF[3], _SIN_COEF[2], _SIN_COEF[1],
              _SIN_COEF[0]):
        p = p * u + c
    return r * p


def _posenc(knn, freqs, phase):
    P_ = knn.shape[0]
    xb = jnp.concatenate(
        [jnp.broadcast_to(knn[:, c:c + 1], (P_, 20)) for c in range(3)], axis=1)
    xf = xb * freqs
    return _fast_sin(xf + phase)


def _enc_consts(dtype):
    j = jax.lax.broadcasted_iota(jnp.int32, (1, CENC), 1)
    freqs = jnp.round(jnp.exp(0.6931471805599453 * ((j % 20) % L).astype(dtype)))
    phase = jnp.where((j % 20) >= L, dtype(1.5707963267948966), dtype(0.0))
    return freqs, phase


def _bn_fold(W, gram_in, mu_in, b_row, g_row, be_row):
    m0 = _dot(mu_in, W)
    t = _dot(gram_in, W)
    q = jnp.sum(_transpose(W) * t, axis=0, keepdims=True)
    var = q - m0 * m0
    mean = m0 + b_row
    s = g_row / jnp.sqrt(var + EPS)
    c = (b_row - mean) * s + be_row
    return s, c


def _pass_a_kernel(posT_ref, seed_ref, pos256_ref,
                   W1_ref, b1_ref, g1_ref, be1_ref,
                   W5_ref, b5_ref, g5_ref, be5_ref,
                   rel_ref, penc_ref, relm_ref, pencm_ref,
                   W1s_ref, c1_ref, W5s_ref, c5_ref):
    b = pl.program_id(0)
    i = pl.program_id(1)
    q = posT_ref[0]
    s = seed_ref[0]
    p256 = pos256_ref[0]

    qb = q.astype(_BF16).astype(_F32)
    sb = s.astype(_BF16).astype(_F32)
    nq = (q[:, 0:1] * q[:, 0:1] + q[:, 1:2] * q[:, 1:2]) + q[:, 2:3] * q[:, 2:3]
    ns = (s[0:1, :] * s[0:1, :] + s[1:2, :] * s[1:2, :]) + s[2:3, :] * s[2:3, :]
    dot = ((qb[:, 0:1] * sb[0:1, :] + qb[:, 1:2] * sb[1:2, :])
           + qb[:, 2:3] * sb[2:3, :])
    d = (nq - 2.0 * dot) + ns

    iota = jax.lax.broadcasted_iota(jnp.int32, (NB_A, M), 1).astype(_F32)
    oh_parts = []
    for _ in range(K):
        mn = jnp.min(d, axis=1, keepdims=True)
        idx = jnp.min(jnp.where(d == mn, iota, _F32(M)), axis=1, keepdims=True)
        oh_parts.append((iota == idx).astype(_F32))
        d = jnp.where(iota == idx, jnp.inf, d)

    oh_all = jnp.concatenate(oh_parts, axis=0)
    knn = _gather_dot(oh_all, p256)
    qt = jnp.concatenate([q] * K, axis=0)
    rel = qt - knn
    rel_ref[...] = rel

    freqs, phase = _enc_consts(_F32)
    penc = _posenc(knn, freqs, phase)
    penc_ref[...] = penc

    rel_sum = jnp.sum(rel, axis=0, keepdims=True)
    rel_gram = jax.lax.dot_general(rel, rel, (((0,), (0,)), ((), ())),
                                   preferred_element_type=_F32)
    penc_sum = jnp.sum(penc, axis=0, keepdims=True)
    penc_gram = jax.lax.dot_general(penc, penc, (((0,), (0,)), ((), ())),
                                    preferred_element_type=_F32)

    relm = jnp.concatenate([rel_sum, rel_gram], axis=0)
    pencm = jnp.concatenate([penc_sum, penc_gram], axis=0)

    @pl.when((b == 0) & (i == 0))
    def _():
        relm_ref[...] = relm
        pencm_ref[...] = pencm

    @pl.when((b > 0) | (i > 0))
    def _():
        relm_ref[...] += relm
        pencm_ref[...] += pencm

    @pl.when((b == B - 1) & (i == (N // NB_A) - 1))
    def _():
        s1, c1 = _bn_fold(W1_ref[...], relm_ref[1:4, :] * _INVP,
                          relm_ref[0:1, :] * _INVP,
                          b1_ref[...], g1_ref[...], be1_ref[...])
        W1s_ref[...] = W1_ref[...] * _transpose(s1)
        c1_ref[...] = c1
        s5, c5 = _bn_fold(W5_ref[...], pencm_ref[1:61, :] * _INVP,
                          pencm_ref[0:1, :] * _INVP,
                          b5_ref[...], g5_ref[...], be5_ref[...])
        W5s_ref[...] = W5_ref[...] * _transpose(s5)
        c5_ref[...] = c5


def _pass_b_kernel(rel_ref, W1s_ref, c1_ref,
                   W2_ref, b2_ref, W3_ref, b3_ref, g3_ref, be3_ref,
                   a1m_ref, W3s_ref, c3_ref):
    i = pl.program_id(0)
    a1 = jnp.maximum(_dot(rel_ref[...], W1s_ref[...]) + c1_ref[...], 0.0)
    a1_sum = jnp.sum(a1, axis=0, keepdims=True)
    a1_gram = jax.lax.dot_general(a1, a1, (((0,), (0,)), ((), ())),
                                  preferred_element_type=_F32)
    a1m = jnp.concatenate([a1_sum, a1_gram], axis=0)

    @pl.when(i == 0)
    def _():
        a1m_ref[...] = a1m

    @pl.when(i > 0)
    def _():
        a1m_ref[...] += a1m

    @pl.when(i == (P_TOT // PB_B) - 1)
    def _():
        Mw = jax.lax.dot_general(W3_ref[...], W2_ref[...],
                                 (((1,), (0,)), ((), ())),
                                 preferred_element_type=_F32)
        b_row = _dot(b2_ref[...], W3_ref[...]) + b3_ref[...]
        s3, c3 = _bn_fold(Mw, a1m_ref[1:65, :] * _INVP,
                          a1m_ref[0:1, :] * _INVP,
                          b_row, g3_ref[...], be3_ref[...])
        W3s_ref[...] = W3_ref[...] * _transpose(s3)
        c3_ref[...] = c3


def _pass_c_kernel(rel_ref, penc_ref,
                   W1s_ref, c1_ref, W2_ref, b2_ref,
                   W3s_ref, c3_ref, W4_ref, b4_ref,
                   W5s_ref, c5_ref, W6_ref, b6_ref,
                   out_ref):
    rel = rel_ref[...]
    penc = penc_ref[...]

    a1 = jnp.maximum(_dot(rel, W1s_ref[...]) + c1_ref[...], 0.0)
    pe = _dot(a1, W2_ref[...]) + b2_ref[...]
    w3 = jnp.maximum(_dot(pe, W3s_ref[...]) + c3_ref[...], 0.0)
    w4 = _dot(w3, W4_ref[...]) + b4_ref[...]

    f5 = jnp.maximum(_dot(penc, W5s_ref[...]) + c5_ref[...], 0.0)
    f6 = _dot(f5, W6_ref[...]) + b6_ref[...] + pe

    wk = [w4[k * NB_C:(k + 1) * NB_C, :] for k in range(K)]
    mx = wk[0]
    for k in range(1, K):
        mx = jnp.maximum(mx, wk[k])
    ek = [jnp.exp(wk[k] - mx) for k in range(K)]
    den = ek[0]
    for k in range(1, K):
        den = den + ek[k]
    acc = jnp.zeros((NB_C, 128), _F32)
    for k in range(K):
        acc = acc + ek[k] * f6[k * NB_C:(k + 1) * NB_C, :]
    out_ref[...] = acc / den


@functools.partial(jax.jit, static_argnums=())
def kernel(pos, seed, W1, b1, g1, be1, W2, b2, W3, b3, g3, be3, W4, b4,
           W5, b5, g5, be5, W6, b6):
    nbn = N // NB_A
    posT = pos.transpose(0, 2, 1)
    pos256 = pos[:, :, :M]
    row = lambda v: v[None, :]

    full = lambda shape: pl.BlockSpec(shape, lambda b, i: tuple(0 for _ in shape))

    rel, penc, relm, pencm, W1s, c1, W5s, c5 = pl.pallas_call(
        _pass_a_kernel,
        grid=(B, nbn),
        in_specs=[
            pl.BlockSpec((1, NB_A, 3), lambda b, i: (b, i, 0)),
            pl.BlockSpec((1, 3, M), lambda b, i: (b, 0, 0)),
            pl.BlockSpec((1, 3, M), lambda b, i: (b, 0, 0)),
            full((64, 3)), full((1, 64)), full((1, 64)), full((1, 64)),
            full((128, CENC)), full((1, 128)), full((1, 128)), full((1, 128)),
        ],
        out_specs=[
            pl.BlockSpec((K * NB_A, 3), lambda b, i: (b * (N // NB_A) + i, 0)),
            pl.BlockSpec((K * NB_A, CENC), lambda b, i: (b * (N // NB_A) + i, 0)),
            full((4, 3)), full((61, CENC)),
            full((64, 3)), full((1, 64)), full((128, CENC)), full((1, 128)),
        ],
        out_shape=[
            jax.ShapeDtypeStruct((P_TOT, 3), _F32),
            jax.ShapeDtypeStruct((P_TOT, CENC), _F32),
            jax.ShapeDtypeStruct((4, 3), _F32),
            jax.ShapeDtypeStruct((61, CENC), _F32),
            jax.ShapeDtypeStruct((64, 3), _F32),
            jax.ShapeDtypeStruct((1, 64), _F32),
            jax.ShapeDtypeStruct((128, CENC), _F32),
            jax.ShapeDtypeStruct((1, 128), _F32),
        ],
    )(posT, seed, pos256, W1, row(b1), row(g1), row(be1),
      W5, row(b5), row(g5), row(be5))

    fullb = lambda shape: pl.BlockSpec(shape, lambda i: tuple(0 for _ in shape))
    a1m, W3s, c3 = pl.pallas_call(
        _pass_b_kernel,
        grid=(P_TOT // PB_B,),
        in_specs=[
            pl.BlockSpec((PB_B, 3), lambda i: (i, 0)),
            fullb((64, 3)), fullb((1, 64)),
            fullb((128, 64)), fullb((1, 128)),
            fullb((512, 128)), fullb((1, 512)), fullb((1, 512)), fullb((1, 512)),
        ],
        out_specs=[fullb((65, 64)), fullb((512, 128)), fullb((1, 512))],
        out_shape=[
            jax.ShapeDtypeStruct((65, 64), _F32),
            jax.ShapeDtypeStruct((512, 128), _F32),
            jax.ShapeDtypeStruct((1, 512), _F32),
        ],
    )(rel, W1s, c1, W2, row(b2), W3, row(b3), row(g3), row(be3))

    out = pl.pallas_call(
        _pass_c_kernel,
        grid=(P_TOT // PB_C,),
        in_specs=[
            pl.BlockSpec((PB_C, 3), lambda i: (i, 0)),
            pl.BlockSpec((PB_C, CENC), lambda i: (i, 0)),
            fullb((64, 3)), fullb((1, 64)),
            fullb((128, 64)), fullb((1, 128)),
            fullb((512, 128)), fullb((1, 512)),
            fullb((128, 512)), fullb((1, 128)),
            fullb((128, CENC)), fullb((1, 128)),
            fullb((128, 128)), fullb((1, 128)),
        ],
        out_specs=pl.BlockSpec((NB_C, 128), lambda i: (i, 0)),
        out_shape=jax.ShapeDtypeStruct((B * N, 128), _F32),
    )(rel, penc, W1s, c1, W2, row(b2), W3s, c3, W4, row(b4),
      W5s, c5, W6, row(b6))

    return out.reshape(B, N, 128).transpose(0, 2, 1)


# speedup vs baseline: 1.1211x; 1.0122x over previous
import functools

import jax
import jax.numpy as jnp
from jax.experimental import pallas as pl

B = 4
N = 2048
M = 256
K = 8
CENC = 60
L = 10
EPS = 1e-5

NB_A = 512
PB_B = 8192
NB_C = 512
PB_C = NB_C * K
P_TOT = B * N * K

_F32 = jnp.float32
_BF16 = jnp.bfloat16
_INVP = 1.0 / P_TOT


def _dot(x, w):
    return jax.lax.dot_general(x, w, (((1,), (1,)), ((), ())),
                               preferred_element_type=_F32)


def _gather_dot(oh, w):
    hi = w.astype(_BF16).astype(_F32)
    r = w - hi
    mid = r.astype(_BF16).astype(_F32)
    lo = r - mid
    return _dot(oh, hi) + _dot(oh, mid) + _dot(oh, lo)


def _eye(n):
    r = jax.lax.broadcasted_iota(jnp.int32, (n, n), 0)
    c = jax.lax.broadcasted_iota(jnp.int32, (n, n), 1)
    return (r == c).astype(_F32)


def _transpose(x):
    return _gather_dot(_eye(x.shape[1]), x)


_INV2PI = 0.15915493667125702
_C1 = 6.28125
_C2 = 0.0019353071693331003
_C3 = 1.0253131677018246e-11
_SIN_COEF = (0.9999995827674866, -0.1666654646396637, 0.008332370780408382,
             -0.00019807845819741488, 2.69886936621333e-06,
             -2.03291836697872e-08)


def _fast_sin(x):
    n = jnp.floor(x * _INV2PI + 0.5)
    r = ((x - n * _C1) - n * _C2) - n * _C3
    u = r * r
    p = _F32(_SIN_COEF[5])
    for c in (_SIN_COEF[4], _SIN_COEF[3], _SIN_COEF[2], _SIN_COEF[1],
              _SIN_COEF[0]):
        p = p * u + c
    return r * p


def _posenc(knn, freqs, phase):
    P_ = knn.shape[0]
    xb = jnp.concatenate(
        [jnp.broadcast_to(knn[:, c:c + 1], (P_, 20)) for c in range(3)], axis=1)
    xf = xb * freqs
    return _fast_sin(xf + phase)


def _enc_consts(dtype):
    j = jax.lax.broadcasted_iota(jnp.int32, (1, CENC), 1)
    freqs = jnp.round(jnp.exp(0.6931471805599453 * ((j % 20) % L).astype(dtype)))
    phase = jnp.where((j % 20) >= L, dtype(1.5707963267948966), dtype(0.0))
    return freqs, phase


def _bn_fold(W, gram_in, mu_in, b_row, g_row, be_row):
    m0 = _dot(mu_in, W)
    t = _dot(gram_in, W)
    q = jnp.sum(_transpose(W) * t, axis=0, keepdims=True)
    var = q - m0 * m0
    mean = m0 + b_row
    s = g_row / jnp.sqrt(var + EPS)
    c = (b_row - mean) * s + be_row
    return s, c


def _pass_a_kernel(posT_ref, seed_ref, pos256_ref,
                   W1_ref, b1_ref, g1_ref, be1_ref,
                   W5_ref, b5_ref, g5_ref, be5_ref,
                   rel_ref, penc_ref, relm_ref, pencm_ref,
                   W1s_ref, c1_ref, W5s_ref, c5_ref):
    b = pl.program_id(0)
    i = pl.program_id(1)
    q = posT_ref[0]
    s = seed_ref[0]
    p256 = pos256_ref[0]

    qb = q.astype(_BF16).astype(_F32)
    sb = s.astype(_BF16).astype(_F32)
    nq = (q[:, 0:1] * q[:, 0:1] + q[:, 1:2] * q[:, 1:2]) + q[:, 2:3] * q[:, 2:3]
    ns = (s[0:1, :] * s[0:1, :] + s[1:2, :] * s[1:2, :]) + s[2:3, :] * s[2:3, :]
    dot = ((qb[:, 0:1] * sb[0:1, :] + qb[:, 1:2] * sb[1:2, :])
           + qb[:, 2:3] * sb[2:3, :])
    d = (nq - 2.0 * dot) + ns

    iota = jax.lax.broadcasted_iota(jnp.int32, (NB_A, M), 1).astype(_F32)
    oh_parts = []
    for _ in range(K):
        mn = jnp.min(d, axis=1, keepdims=True)
        idx = jnp.min(jnp.where(d == mn, iota, _F32(M)), axis=1, keepdims=True)
        oh_parts.append((iota == idx).astype(_F32))
        d = jnp.where(iota == idx, jnp.inf, d)

    oh_all = jnp.concatenate(oh_parts, axis=0)
    knn = _gather_dot(oh_all, p256)
    qt = jnp.concatenate([q] * K, axis=0)
    rel = qt - knn
    rel_ref[...] = rel.astype(_BF16)

    freqs, phase = _enc_consts(_F32)
    penc = _posenc(knn, freqs, phase)
    penc_ref[...] = penc.astype(_BF16)

    rel_sum = jnp.sum(rel, axis=0, keepdims=True)
    rel_gram = jax.lax.dot_general(rel, rel, (((0,), (0,)), ((), ())),
                                   preferred_element_type=_F32)
    penc_sum = jnp.sum(penc, axis=0, keepdims=True)
    penc_gram = jax.lax.dot_general(penc, penc, (((0,), (0,)), ((), ())),
                                    preferred_element_type=_F32)

    relm = jnp.concatenate([rel_sum, rel_gram], axis=0)
    pencm = jnp.concatenate([penc_sum, penc_gram], axis=0)

    @pl.when((b == 0) & (i == 0))
    def _():
        relm_ref[...] = relm
        pencm_ref[...] = pencm

    @pl.when((b > 0) | (i > 0))
    def _():
        relm_ref[...] += relm
        pencm_ref[...] += pencm

    @pl.when((b == B - 1) & (i == (N // NB_A) - 1))
    def _():
        s1, c1 = _bn_fold(W1_ref[...], relm_ref[1:4, :] * _INVP,
                          relm_ref[0:1, :] * _INVP,
                          b1_ref[...], g1_ref[...], be1_ref[...])
        W1s_ref[...] = W1_ref[...] * _transpose(s1)
        c1_ref[...] = c1
        s5, c5 = _bn_fold(W5_ref[...], pencm_ref[1:61, :] * _INVP,
                          pencm_ref[0:1, :] * _INVP,
                          b5_ref[...], g5_ref[...], be5_ref[...])
        W5s_ref[...] = W5_ref[...] * _transpose(s5)
        c5_ref[...] = c5


def _pass_b_kernel(rel_ref, W1s_ref, c1_ref,
                   W2_ref, b2_ref, W3_ref, b3_ref, g3_ref, be3_ref,
                   a1m_ref, W3s_ref, c3_ref):
    i = pl.program_id(0)
    a1 = jnp.maximum(_dot(rel_ref[...].astype(_F32), W1s_ref[...])
                     + c1_ref[...], 0.0)
    a1_sum = jnp.sum(a1, axis=0, keepdims=True)
    a1_gram = jax.lax.dot_general(a1, a1, (((0,), (0,)), ((), ())),
                                  preferred_element_type=_F32)
    a1m = jnp.concatenate([a1_sum, a1_gram], axis=0)

    @pl.when(i == 0)
    def _():
        a1m_ref[...] = a1m

    @pl.when(i > 0)
    def _():
        a1m_ref[...] += a1m

    @pl.when(i == (P_TOT // PB_B) - 1)
    def _():
        Mw = jax.lax.dot_general(W3_ref[...], W2_ref[...],
                                 (((1,), (0,)), ((), ())),
                                 preferred_element_type=_F32)
        b_row = _dot(b2_ref[...], W3_ref[...]) + b3_ref[...]
        s3, c3 = _bn_fold(Mw, a1m_ref[1:65, :] * _INVP,
                          a1m_ref[0:1, :] * _INVP,
                          b_row, g3_ref[...], be3_ref[...])
        W3s_ref[...] = W3_ref[...] * _transpose(s3)
        c3_ref[...] = c3


def _pass_c_kernel(rel_ref, penc_ref,
                   W1s_ref, c1_ref, W2_ref, b2_ref,
                   W3s_ref, c3_ref, W4_ref, b4_ref,
                   W5s_ref, c5_ref, W6_ref, b6_ref,
                   out_ref):
    rel = rel_ref[...].astype(_F32)
    penc = penc_ref[...].astype(_F32)

    a1 = jnp.maximum(_dot(rel, W1s_ref[...]) + c1_ref[...], 0.0)
    pe = _dot(a1, W2_ref[...]) + b2_ref[...]
    w3 = jnp.maximum(_dot(pe, W3s_ref[...]) + c3_ref[...], 0.0)
    w4 = _dot(w3, W4_ref[...]) + b4_ref[...]

    f5 = jnp.maximum(_dot(penc, W5s_ref[...]) + c5_ref[...], 0.0)
    f6 = _dot(f5, W6_ref[...]) + b6_ref[...] + pe

    wk = [w4[k * NB_C:(k + 1) * NB_C, :] for k in range(K)]
    mx = wk[0]
    for k in range(1, K):
        mx = jnp.maximum(mx, wk[k])
    ek = [jnp.exp(wk[k] - mx) for k in range(K)]
    den = ek[0]
    for k in range(1, K):
        den = den + ek[k]
    acc = jnp.zeros((NB_C, 128), _F32)
    for k in range(K):
        acc = acc + ek[k] * f6[k * NB_C:(k + 1) * NB_C, :]
    out_ref[...] = acc / den


@functools.partial(jax.jit, static_argnums=())
def kernel(pos, seed, W1, b1, g1, be1, W2, b2, W3, b3, g3, be3, W4, b4,
           W5, b5, g5, be5, W6, b6):
    nbn = N // NB_A
    posT = pos.transpose(0, 2, 1)
    pos256 = pos[:, :, :M]
    row = lambda v: v[None, :]

    full = lambda shape: pl.BlockSpec(shape, lambda b, i: tuple(0 for _ in shape))

    rel, penc, relm, pencm, W1s, c1, W5s, c5 = pl.pallas_call(
        _pass_a_kernel,
        grid=(B, nbn),
        in_specs=[
            pl.BlockSpec((1, NB_A, 3), lambda b, i: (b, i, 0)),
            pl.BlockSpec((1, 3, M), lambda b, i: (b, 0, 0)),
            pl.BlockSpec((1, 3, M), lambda b, i: (b, 0, 0)),
            full((64, 3)), full((1, 64)), full((1, 64)), full((1, 64)),
            full((128, CENC)), full((1, 128)), full((1, 128)), full((1, 128)),
        ],
        out_specs=[
            pl.BlockSpec((K * NB_A, 3), lambda b, i: (b * (N // NB_A) + i, 0)),
            pl.BlockSpec((K * NB_A, CENC), lambda b, i: (b * (N // NB_A) + i, 0)),
            full((4, 3)), full((61, CENC)),
            full((64, 3)), full((1, 64)), full((128, CENC)), full((1, 128)),
        ],
        out_shape=[
            jax.ShapeDtypeStruct((P_TOT, 3), _BF16),
            jax.ShapeDtypeStruct((P_TOT, CENC), _BF16),
            jax.ShapeDtypeStruct((4, 3), _F32),
            jax.ShapeDtypeStruct((61, CENC), _F32),
            jax.ShapeDtypeStruct((64, 3), _F32),
            jax.ShapeDtypeStruct((1, 64), _F32),
            jax.ShapeDtypeStruct((128, CENC), _F32),
            jax.ShapeDtypeStruct((1, 128), _F32),
        ],
    )(posT, seed, pos256, W1, row(b1), row(g1), row(be1),
      W5, row(b5), row(g5), row(be5))

    fullb = lambda shape: pl.BlockSpec(shape, lambda i: tuple(0 for _ in shape))
    a1m, W3s, c3 = pl.pallas_call(
        _pass_b_kernel,
        grid=(P_TOT // PB_B,),
        in_specs=[
            pl.BlockSpec((PB_B, 3), lambda i: (i, 0)),
            fullb((64, 3)), fullb((1, 64)),
            fullb((128, 64)), fullb((1, 128)),
            fullb((512, 128)), fullb((1, 512)), fullb((1, 512)), fullb((1, 512)),
        ],
        out_specs=[fullb((65, 64)), fullb((512, 128)), fullb((1, 512))],
        out_shape=[
            jax.ShapeDtypeStruct((65, 64), _F32),
            jax.ShapeDtypeStruct((512, 128), _F32),
            jax.ShapeDtypeStruct((1, 512), _F32),
        ],
    )(rel, W1s, c1, W2, row(b2), W3, row(b3), row(g3), row(be3))

    out = pl.pallas_call(
        _pass_c_kernel,
        grid=(P_TOT // PB_C,),
        in_specs=[
            pl.BlockSpec((PB_C, 3), lambda i: (i, 0)),
            pl.BlockSpec((PB_C, CENC), lambda i: (i, 0)),
            fullb((64, 3)), fullb((1, 64)),
            fullb((128, 64)), fullb((1, 128)),
            fullb((512, 128)), fullb((1, 512)),
            fullb((128, 512)), fullb((1, 128)),
            fullb((128, CENC)), fullb((1, 128)),
            fullb((128, 128)), fullb((1, 128)),
        ],
        out_specs=pl.BlockSpec((NB_C, 128), lambda i: (i, 0)),
        out_shape=jax.ShapeDtypeStruct((B * N, 128), _F32),
    )(rel, penc, W1s, c1, W2, row(b2), W3s, c3, W4, row(b4),
      W5s, c5, W6, row(b6))

    return out.reshape(B, N, 128).transpose(0, 2, 1)


# speedup vs baseline: 1.1214x; 1.0002x over previous
import functools

import jax
import jax.numpy as jnp
from jax.experimental import pallas as pl

B = 4
N = 2048
M = 256
K = 8
CENC = 60
L = 10
EPS = 1e-5

NB_A = 512
PB_B = 32768
NB_C = 512
PB_C = NB_C * K
P_TOT = B * N * K

_F32 = jnp.float32
_BF16 = jnp.bfloat16
_INVP = 1.0 / P_TOT


def _dot(x, w):
    return jax.lax.dot_general(x, w, (((1,), (1,)), ((), ())),
                               preferred_element_type=_F32)


def _gather_dot(oh, w):
    hi = w.astype(_BF16).astype(_F32)
    r = w - hi
    mid = r.astype(_BF16).astype(_F32)
    lo = r - mid
    return _dot(oh, hi) + _dot(oh, mid) + _dot(oh, lo)


def _eye(n):
    r = jax.lax.broadcasted_iota(jnp.int32, (n, n), 0)
    c = jax.lax.broadcasted_iota(jnp.int32, (n, n), 1)
    return (r == c).astype(_F32)


def _transpose(x):
    return _gather_dot(_eye(x.shape[1]), x)


_INV2PI = 0.15915493667125702
_C1 = 6.28125
_C2 = 0.0019353071693331003
_C3 = 1.0253131677018246e-11
_SIN_COEF = (0.9999995827674866, -0.1666654646396637, 0.008332370780408382,
             -0.00019807845819741488, 2.69886936621333e-06,
             -2.03291836697872e-08)


def _fast_sin(x):
    n = jnp.floor(x * _INV2PI + 0.5)
    r = ((x - n * _C1) - n * _C2) - n * _C3
    u = r * r
    p = _F32(_SIN_COEF[5])
    for c in (_SIN_COEF[4], _SIN_COEF[3], _SIN_COEF[2], _SIN_COEF[1],
              _SIN_COEF[0]):
        p = p * u + c
    return r * p


def _posenc(knn, freqs, phase):
    P_ = knn.shape[0]
    xb = jnp.concatenate(
        [jnp.broadcast_to(knn[:, c:c + 1], (P_, 20)) for c in range(3)], axis=1)
    xf = xb * freqs
    return _fast_sin(xf + phase)


def _enc_consts(dtype):
    j = jax.lax.broadcasted_iota(jnp.int32, (1, CENC), 1)
    freqs = jnp.round(jnp.exp(0.6931471805599453 * ((j % 20) % L).astype(dtype)))
    phase = jnp.where((j % 20) >= L, dtype(1.5707963267948966), dtype(0.0))
    return freqs, phase


def _bn_fold(W, gram_in, mu_in, b_row, g_row, be_row):
    m0 = _dot(mu_in, W)
    t = _dot(gram_in, W)
    q = jnp.sum(_transpose(W) * t, axis=0, keepdims=True)
    var = q - m0 * m0
    mean = m0 + b_row
    s = g_row / jnp.sqrt(var + EPS)
    c = (b_row - mean) * s + be_row
    return s, c


def _pass_a_kernel(posT_ref, seed_ref, pos256_ref,
                   W1_ref, b1_ref, g1_ref, be1_ref,
                   W5_ref, b5_ref, g5_ref, be5_ref,
                   rel_ref, penc_ref, relm_ref, pencm_ref,
                   W1s_ref, c1_ref, W5s_ref, c5_ref):
    b = pl.program_id(0)
    i = pl.program_id(1)
    q = posT_ref[0]
    s = seed_ref[0]
    p256 = pos256_ref[0]

    qb = q.astype(_BF16).astype(_F32)
    sb = s.astype(_BF16).astype(_F32)
    nq = (q[:, 0:1] * q[:, 0:1] + q[:, 1:2] * q[:, 1:2]) + q[:, 2:3] * q[:, 2:3]
    ns = (s[0:1, :] * s[0:1, :] + s[1:2, :] * s[1:2, :]) + s[2:3, :] * s[2:3, :]
    dot = ((qb[:, 0:1] * sb[0:1, :] + qb[:, 1:2] * sb[1:2, :])
           + qb[:, 2:3] * sb[2:3, :])
    d = (nq - 2.0 * dot) + ns

    iota = jax.lax.broadcasted_iota(jnp.int32, (NB_A, M), 1).astype(_F32)
    oh_parts = []
    for _ in range(K):
        mn = jnp.min(d, axis=1, keepdims=True)
        idx = jnp.min(jnp.where(d == mn, iota, _F32(M)), axis=1, keepdims=True)
        oh_parts.append((iota == idx).astype(_F32))
        d = jnp.where(iota == idx, jnp.inf, d)

    oh_all = jnp.concatenate(oh_parts, axis=0)
    knn = _gather_dot(oh_all, p256)
    qt = jnp.concatenate([q] * K, axis=0)
    rel = qt - knn
    rel_ref[...] = rel.astype(_BF16)

    freqs, phase = _enc_consts(_F32)
    penc = _posenc(knn, freqs, phase)
    penc_ref[...] = penc.astype(_BF16)

    rel_sum = jnp.sum(rel, axis=0, keepdims=True)
    rel_gram = jax.lax.dot_general(rel, rel, (((0,), (0,)), ((), ())),
                                   preferred_element_type=_F32)
    penc_sum = jnp.sum(penc, axis=0, keepdims=True)
    penc_gram = jax.lax.dot_general(penc, penc, (((0,), (0,)), ((), ())),
                                    preferred_element_type=_F32)

    relm = jnp.concatenate([rel_sum, rel_gram], axis=0)
    pencm = jnp.concatenate([penc_sum, penc_gram], axis=0)

    @pl.when((b == 0) & (i == 0))
    def _():
        relm_ref[...] = relm
        pencm_ref[...] = pencm

    @pl.when((b > 0) | (i > 0))
    def _():
        relm_ref[...] += relm
        pencm_ref[...] += pencm

    @pl.when((b == B - 1) & (i == (N // NB_A) - 1))
    def _():
        s1, c1 = _bn_fold(W1_ref[...], relm_ref[1:4, :] * _INVP,
                          relm_ref[0:1, :] * _INVP,
                          b1_ref[...], g1_ref[...], be1_ref[...])
        W1s_ref[...] = W1_ref[...] * _transpose(s1)
        c1_ref[...] = c1
        s5, c5 = _bn_fold(W5_ref[...], pencm_ref[1:61, :] * _INVP,
                          pencm_ref[0:1, :] * _INVP,
                          b5_ref[...], g5_ref[...], be5_ref[...])
        W5s_ref[...] = W5_ref[...] * _transpose(s5)
        c5_ref[...] = c5


def _pass_b_kernel(rel_ref, W1s_ref, c1_ref,
                   W2_ref, b2_ref, W3_ref, b3_ref, g3_ref, be3_ref,
                   a1m_ref, W3s_ref, c3_ref):
    i = pl.program_id(0)
    a1 = jnp.maximum(_dot(rel_ref[...].astype(_F32), W1s_ref[...])
                     + c1_ref[...], 0.0)
    a1_sum = jnp.sum(a1, axis=0, keepdims=True)
    a1_gram = jax.lax.dot_general(a1, a1, (((0,), (0,)), ((), ())),
                                  preferred_element_type=_F32)
    a1m = jnp.concatenate([a1_sum, a1_gram], axis=0)

    @pl.when(i == 0)
    def _():
        a1m_ref[...] = a1m

    @pl.when(i > 0)
    def _():
        a1m_ref[...] += a1m

    @pl.when(i == (P_TOT // PB_B) - 1)
    def _():
        Mw = jax.lax.dot_general(W3_ref[...], W2_ref[...],
                                 (((1,), (0,)), ((), ())),
                                 preferred_element_type=_F32)
        b_row = _dot(b2_ref[...], W3_ref[...]) + b3_ref[...]
        s3, c3 = _bn_fold(Mw, a1m_ref[1:65, :] * _INVP,
                          a1m_ref[0:1, :] * _INVP,
                          b_row, g3_ref[...], be3_ref[...])
        W3s_ref[...] = W3_ref[...] * _transpose(s3)
        c3_ref[...] = c3


def _pass_c_kernel(rel_ref, penc_ref,
                   W1s_ref, c1_ref, W2_ref, b2_ref,
                   W3s_ref, c3_ref, W4_ref, b4_ref,
                   W5s_ref, c5_ref, W6_ref, b6_ref,
                   out_ref):
    rel = rel_ref[...].astype(_F32)
    penc = penc_ref[...].astype(_F32)

    a1 = jnp.maximum(_dot(rel, W1s_ref[...]) + c1_ref[...], 0.0)
    pe = _dot(a1, W2_ref[...]) + b2_ref[...]
    w3 = jnp.maximum(_dot(pe, W3s_ref[...]) + c3_ref[...], 0.0)
    w4 = _dot(w3, W4_ref[...]) + b4_ref[...]

    f5 = jnp.maximum(_dot(penc, W5s_ref[...]) + c5_ref[...], 0.0)
    f6 = _dot(f5, W6_ref[...]) + b6_ref[...] + pe

    wk = [w4[k * NB_C:(k + 1) * NB_C, :] for k in range(K)]
    mx = wk[0]
    for k in range(1, K):
        mx = jnp.maximum(mx, wk[k])
    ek = [jnp.exp(wk[k] - mx) for k in range(K)]
    den = ek[0]
    for k in range(1, K):
        den = den + ek[k]
    acc = jnp.zeros((NB_C, 128), _F32)
    for k in range(K):
        acc = acc + ek[k] * f6[k * NB_C:(k + 1) * NB_C, :]
    out_ref[...] = acc / den


@functools.partial(jax.jit, static_argnums=())
def kernel(pos, seed, W1, b1, g1, be1, W2, b2, W3, b3, g3, be3, W4, b4,
           W5, b5, g5, be5, W6, b6):
    nbn = N // NB_A
    posT = pos.transpose(0, 2, 1)
    pos256 = pos[:, :, :M]
    row = lambda v: v[None, :]

    full = lambda shape: pl.BlockSpec(shape, lambda b, i: tuple(0 for _ in shape))

    rel, penc, relm, pencm, W1s, c1, W5s, c5 = pl.pallas_call(
        _pass_a_kernel,
        grid=(B, nbn),
        in_specs=[
            pl.BlockSpec((1, NB_A, 3), lambda b, i: (b, i, 0)),
            pl.BlockSpec((1, 3, M), lambda b, i: (b, 0, 0)),
            pl.BlockSpec((1, 3, M), lambda b, i: (b, 0, 0)),
            full((64, 3)), full((1, 64)), full((1, 64)), full((1, 64)),
            full((128, CENC)), full((1, 128)), full((1, 128)), full((1, 128)),
        ],
        out_specs=[
            pl.BlockSpec((K * NB_A, 3), lambda b, i: (b * (N // NB_A) + i, 0)),
            pl.BlockSpec((K * NB_A, CENC), lambda b, i: (b * (N // NB_A) + i, 0)),
            full((4, 3)), full((61, CENC)),
            full((64, 3)), full((1, 64)), full((128, CENC)), full((1, 128)),
        ],
        out_shape=[
            jax.ShapeDtypeStruct((P_TOT, 3), _BF16),
            jax.ShapeDtypeStruct((P_TOT, CENC), _BF16),
            jax.ShapeDtypeStruct((4, 3), _F32),
            jax.ShapeDtypeStruct((61, CENC), _F32),
            jax.ShapeDtypeStruct((64, 3), _F32),
            jax.ShapeDtypeStruct((1, 64), _F32),
            jax.ShapeDtypeStruct((128, CENC), _F32),
            jax.ShapeDtypeStruct((1, 128), _F32),
        ],
    )(posT, seed, pos256, W1, row(b1), row(g1), row(be1),
      W5, row(b5), row(g5), row(be5))

    fullb = lambda shape: pl.BlockSpec(shape, lambda i: tuple(0 for _ in shape))
    a1m, W3s, c3 = pl.pallas_call(
        _pass_b_kernel,
        grid=(P_TOT // PB_B,),
        in_specs=[
            pl.BlockSpec((PB_B, 3), lambda i: (i, 0)),
            fullb((64, 3)), fullb((1, 64)),
            fullb((128, 64)), fullb((1, 128)),
            fullb((512, 128)), fullb((1, 512)), fullb((1, 512)), fullb((1, 512)),
        ],
        out_specs=[fullb((65, 64)), fullb((512, 128)), fullb((1, 512))],
        out_shape=[
            jax.ShapeDtypeStruct((65, 64), _F32),
            jax.ShapeDtypeStruct((512, 128), _F32),
            jax.ShapeDtypeStruct((1, 512), _F32),
        ],
    )(rel, W1s, c1, W2, row(b2), W3, row(b3), row(g3), row(be3))

    out = pl.pallas_call(
        _pass_c_kernel,
        grid=(P_TOT // PB_C,),
        in_specs=[
            pl.BlockSpec((PB_C, 3), lambda i: (i, 0)),
            pl.BlockSpec((PB_C, CENC), lambda i: (i, 0)),
            fullb((64, 3)), fullb((1, 64)),
            fullb((128, 64)), fullb((1, 128)),
            fullb((512, 128)), fullb((1, 512)),
            fullb((128, 512)), fullb((1, 128)),
            fullb((128, CENC)), fullb((1, 128)),
            fullb((128, 128)), fullb((1, 128)),
        ],
        out_specs=pl.BlockSpec((NB_C, 128), lambda i: (i, 0)),
        out_shape=jax.ShapeDtypeStruct((B * N, 128), _F32),
    )(rel, penc, W1s, c1, W2, row(b2), W3s, c3, W4, row(b4),
      W5s, c5, W6, row(b6))

    return out.reshape(B, N, 128).transpose(0, 2, 1)


# speedup vs baseline: 1.1289x; 1.0068x over previous
import functools

import jax
import jax.numpy as jnp
from jax.experimental import pallas as pl
from jax.experimental.pallas import tpu as pltpu

B = 4
N = 2048
M = 256
K = 8
CENC = 60
L = 10
EPS = 1e-5

NB_A = 512
PB_B = 32768
NB_C = 512
PB_C = NB_C * K
P_TOT = B * N * K

_F32 = jnp.float32
_BF16 = jnp.bfloat16
_INVP = 1.0 / P_TOT


def _dot(x, w):
    return jax.lax.dot_general(x, w, (((1,), (1,)), ((), ())),
                               preferred_element_type=_F32)


def _gather_dot(oh, w):
    hi = w.astype(_BF16).astype(_F32)
    r = w - hi
    mid = r.astype(_BF16).astype(_F32)
    lo = r - mid
    return _dot(oh, hi) + _dot(oh, mid) + _dot(oh, lo)


def _eye(n):
    r = jax.lax.broadcasted_iota(jnp.int32, (n, n), 0)
    c = jax.lax.broadcasted_iota(jnp.int32, (n, n), 1)
    return (r == c).astype(_F32)


def _transpose(x):
    return _gather_dot(_eye(x.shape[1]), x)


_INV2PI = 0.15915493667125702
_C1 = 6.28125
_C2 = 0.0019353071693331003
_C3 = 1.0253131677018246e-11
_SIN_COEF = (0.9999995827674866, -0.1666654646396637, 0.008332370780408382,
             -0.00019807845819741488, 2.69886936621333e-06,
             -2.03291836697872e-08)


def _fast_sin(x):
    n = jnp.floor(x * _INV2PI + 0.5)
    r = ((x - n * _C1) - n * _C2) - n * _C3
    u = r * r
    p = _F32(_SIN_COEF[5])
    for c in (_SIN_COEF[4], _SIN_COEF[3], _SIN_COEF[2], _SIN_COEF[1],
              _SIN_COEF[0]):
        p = p * u + c
    return r * p


def _posenc(knn, freqs, phase):
    P_ = knn.shape[0]
    xb = jnp.concatenate(
        [jnp.broadcast_to(knn[:, c:c + 1], (P_, 20)) for c in range(3)], axis=1)
    xf = xb * freqs
    return _fast_sin(xf + phase)


def _enc_consts(dtype):
    j = jax.lax.broadcasted_iota(jnp.int32, (1, CENC), 1)
    freqs = jnp.round(jnp.exp(0.6931471805599453 * ((j % 20) % L).astype(dtype)))
    phase = jnp.where((j % 20) >= L, dtype(1.5707963267948966), dtype(0.0))
    return freqs, phase


def _bn_fold(W, gram_in, mu_in, b_row, g_row, be_row):
    m0 = _dot(mu_in, W)
    t = _dot(gram_in, W)
    q = jnp.sum(_transpose(W) * t, axis=0, keepdims=True)
    var = q - m0 * m0
    mean = m0 + b_row
    s = g_row / jnp.sqrt(var + EPS)
    c = (b_row - mean) * s + be_row
    return s, c


def _pass_a_kernel(posT_ref, seed_ref, pos256_ref,
                   W1_ref, b1_ref, g1_ref, be1_ref,
                   W5_ref, b5_ref, g5_ref, be5_ref,
                   rel_ref, penc_ref, relm_ref, pencm_ref,
                   W1s_ref, c1_ref, W5s_ref, c5_ref):
    b = pl.program_id(0)
    i = pl.program_id(1)
    q = posT_ref[0]
    s = seed_ref[0]
    p256 = pos256_ref[0]

    qb = q.astype(_BF16).astype(_F32)
    sb = s.astype(_BF16).astype(_F32)
    nq = (q[:, 0:1] * q[:, 0:1] + q[:, 1:2] * q[:, 1:2]) + q[:, 2:3] * q[:, 2:3]
    ns = (s[0:1, :] * s[0:1, :] + s[1:2, :] * s[1:2, :]) + s[2:3, :] * s[2:3, :]
    dot = ((qb[:, 0:1] * sb[0:1, :] + qb[:, 1:2] * sb[1:2, :])
           + qb[:, 2:3] * sb[2:3, :])
    d = (nq - 2.0 * dot) + ns

    iota = jax.lax.broadcasted_iota(jnp.int32, (NB_A, M), 1).astype(_F32)
    oh_parts = []
    for _ in range(K):
        mn = jnp.min(d, axis=1, keepdims=True)
        idx = jnp.min(jnp.where(d == mn, iota, _F32(M)), axis=1, keepdims=True)
        oh_parts.append((iota == idx).astype(_F32))
        d = jnp.where(iota == idx, jnp.inf, d)

    oh_all = jnp.concatenate(oh_parts, axis=0)
    knn = _gather_dot(oh_all, p256)
    qt = jnp.concatenate([q] * K, axis=0)
    rel = qt - knn
    rel_ref[...] = rel.astype(_BF16)

    freqs, phase = _enc_consts(_F32)
    penc = _posenc(knn, freqs, phase)
    penc_ref[...] = penc.astype(_BF16)

    rel_sum = jnp.sum(rel, axis=0, keepdims=True)
    rel_gram = jax.lax.dot_general(rel, rel, (((0,), (0,)), ((), ())),
                                   preferred_element_type=_F32)
    penc_sum = jnp.sum(penc, axis=0, keepdims=True)
    penc_gram = jax.lax.dot_general(penc, penc, (((0,), (0,)), ((), ())),
                                    preferred_element_type=_F32)

    relm = jnp.concatenate([rel_sum, rel_gram], axis=0)
    pencm = jnp.concatenate([penc_sum, penc_gram], axis=0)

    @pl.when((b == 0) & (i == 0))
    def _():
        relm_ref[...] = relm
        pencm_ref[...] = pencm

    @pl.when((b > 0) | (i > 0))
    def _():
        relm_ref[...] += relm
        pencm_ref[...] += pencm

    @pl.when((b == B - 1) & (i == (N // NB_A) - 1))
    def _():
        s1, c1 = _bn_fold(W1_ref[...], relm_ref[1:4, :] * _INVP,
                          relm_ref[0:1, :] * _INVP,
                          b1_ref[...], g1_ref[...], be1_ref[...])
        W1s_ref[...] = W1_ref[...] * _transpose(s1)
        c1_ref[...] = c1
        s5, c5 = _bn_fold(W5_ref[...], pencm_ref[1:61, :] * _INVP,
                          pencm_ref[0:1, :] * _INVP,
                          b5_ref[...], g5_ref[...], be5_ref[...])
        W5s_ref[...] = W5_ref[...] * _transpose(s5)
        c5_ref[...] = c5


def _pass_bc_kernel(rel_ref, penc_ref,
                    W1s_ref, c1_ref, W2_ref, b2_ref,
                    W3_ref, b3_ref, g3_ref, be3_ref, W4_ref, b4_ref,
                    W5s_ref, c5_ref, W6_ref, b6_ref,
                    out_ref, a1m_scr, W3s_scr, c3_scr):
    p = pl.program_id(0)
    i = pl.program_id(1)

    @pl.when(p == 0)
    def _():
        a1 = jnp.maximum(_dot(rel_ref[...].astype(_F32), W1s_ref[...])
                         + c1_ref[...], 0.0)
        a1_sum = jnp.sum(a1, axis=0, keepdims=True)
        a1_gram = jax.lax.dot_general(a1, a1, (((0,), (0,)), ((), ())),
                                      preferred_element_type=_F32)
        a1m = jnp.concatenate([a1_sum, a1_gram], axis=0)

        @pl.when(i == 0)
        def _():
            a1m_scr[...] = a1m

        @pl.when(i > 0)
        def _():
            a1m_scr[...] += a1m

        @pl.when(i == (P_TOT // PB_C) - 1)
        def _():
            Mw = jax.lax.dot_general(W3_ref[...], W2_ref[...],
                                     (((1,), (0,)), ((), ())),
                                     preferred_element_type=_F32)
            b_row = _dot(b2_ref[...], W3_ref[...]) + b3_ref[...]
            s3, c3 = _bn_fold(Mw, a1m_scr[1:65, :] * _INVP,
                              a1m_scr[0:1, :] * _INVP,
                              b_row, g3_ref[...], be3_ref[...])
            W3s_scr[...] = W3_ref[...] * _transpose(s3)
            c3_scr[...] = c3

    @pl.when(p == 1)
    def _():
        rel = rel_ref[...].astype(_F32)
        penc = penc_ref[...].astype(_F32)

        a1 = jnp.maximum(_dot(rel, W1s_ref[...]) + c1_ref[...], 0.0)
        pe = _dot(a1, W2_ref[...]) + b2_ref[...]
        w3 = jnp.maximum(_dot(pe, W3s_scr[...]) + c3_scr[...], 0.0)
        w4 = _dot(w3, W4_ref[...]) + b4_ref[...]

        f5 = jnp.maximum(_dot(penc, W5s_ref[...]) + c5_ref[...], 0.0)
        f6 = _dot(f5, W6_ref[...]) + b6_ref[...] + pe

        wk = [w4[k * NB_C:(k + 1) * NB_C, :] for k in range(K)]
        mx = wk[0]
        for k in range(1, K):
            mx = jnp.maximum(mx, wk[k])
        ek = [jnp.exp(wk[k] - mx) for k in range(K)]
        den = ek[0]
        for k in range(1, K):
            den = den + ek[k]
        acc = jnp.zeros((NB_C, 128), _F32)
        for k in range(K):
            acc = acc + ek[k] * f6[k * NB_C:(k + 1) * NB_C, :]
        out_ref[...] = acc / den


@functools.partial(jax.jit, static_argnums=())
def kernel(pos, seed, W1, b1, g1, be1, W2, b2, W3, b3, g3, be3, W4, b4,
           W5, b5, g5, be5, W6, b6):
    nbn = N // NB_A
    posT = pos.transpose(0, 2, 1)
    pos256 = pos[:, :, :M]
    row = lambda v: v[None, :]

    full = lambda shape: pl.BlockSpec(shape, lambda b, i: tuple(0 for _ in shape))

    rel, penc, relm, pencm, W1s, c1, W5s, c5 = pl.pallas_call(
        _pass_a_kernel,
        grid=(B, nbn),
        in_specs=[
            pl.BlockSpec((1, NB_A, 3), lambda b, i: (b, i, 0)),
            pl.BlockSpec((1, 3, M), lambda b, i: (b, 0, 0)),
            pl.BlockSpec((1, 3, M), lambda b, i: (b, 0, 0)),
            full((64, 3)), full((1, 64)), full((1, 64)), full((1, 64)),
            full((128, CENC)), full((1, 128)), full((1, 128)), full((1, 128)),
        ],
        out_specs=[
            pl.BlockSpec((K * NB_A, 3), lambda b, i: (b * (N // NB_A) + i, 0)),
            pl.BlockSpec((K * NB_A, CENC), lambda b, i: (b * (N // NB_A) + i, 0)),
            full((4, 3)), full((61, CENC)),
            full((64, 3)), full((1, 64)), full((128, CENC)), full((1, 128)),
        ],
        out_shape=[
            jax.ShapeDtypeStruct((P_TOT, 3), _BF16),
            jax.ShapeDtypeStruct((P_TOT, CENC), _BF16),
            jax.ShapeDtypeStruct((4, 3), _F32),
            jax.ShapeDtypeStruct((61, CENC), _F32),
            jax.ShapeDtypeStruct((64, 3), _F32),
            jax.ShapeDtypeStruct((1, 64), _F32),
            jax.ShapeDtypeStruct((128, CENC), _F32),
            jax.ShapeDtypeStruct((1, 128), _F32),
        ],
    )(posT, seed, pos256, W1, row(b1), row(g1), row(be1),
      W5, row(b5), row(g5), row(be5))

    fullbc = lambda shape: pl.BlockSpec(
        shape, lambda p, i: tuple(0 for _ in shape))
    nbc = P_TOT // PB_C
    out = pl.pallas_call(
        _pass_bc_kernel,
        grid=(2, nbc),
        in_specs=[
            pl.BlockSpec((PB_C, 3), lambda p, i: (i, 0)),
            pl.BlockSpec((PB_C, CENC), lambda p, i: (p * i, 0)),
            fullbc((64, 3)), fullbc((1, 64)),
            fullbc((128, 64)), fullbc((1, 128)),
            fullbc((512, 128)), fullbc((1, 512)), fullbc((1, 512)),
            fullbc((1, 512)),
            fullbc((128, 512)), fullbc((1, 128)),
            fullbc((128, CENC)), fullbc((1, 128)),
            fullbc((128, 128)), fullbc((1, 128)),
        ],
        out_specs=pl.BlockSpec((NB_C, 128), lambda p, i: (p * i, 0)),
        out_shape=jax.ShapeDtypeStruct((B * N, 128), _F32),
        scratch_shapes=[
            pltpu.VMEM((65, 64), _F32),
            pltpu.VMEM((512, 128), _F32),
            pltpu.VMEM((1, 512), _F32),
        ],
    )(rel, penc, W1s, c1, W2, row(b2), W3, row(b3), row(g3), row(be3),
      W4, row(b4), W5s, c5, W6, row(b6))

    return out.reshape(B, N, 128).transpose(0, 2, 1)
